# baseline (speedup 1.0000x reference)
import functools

import jax
import jax.numpy as jnp
from jax import lax
from jax.experimental import pallas as pl
from jax.experimental.pallas import tpu as pltpu
from jax.experimental.pallas import tpu_sc as plsc

VOCAB = 1000000
EMBED_DIM = 32
BATCH = 16384
HIST = 50
B = BATCH * HIST

NUM_CORES = 2
NUM_SUBCORES = 16
NW = NUM_CORES * NUM_SUBCORES
BPW = B // NW

CHUNK = 1024

_mesh = plsc.VectorSubcoreMesh(core_axis_name="c", subcore_axis_name="s")


def _make_sc_gather(nrw):
    bpw = nrw * CHUNK
    bl = bpw * NW

    @functools.partial(
        pl.kernel,
        out_type=jax.ShapeDtypeStruct((bl // 4, 4, EMBED_DIM), jnp.float32),
        mesh=_mesh,
        scratch_types=[
            pltpu.VMEM((bpw,), jnp.int32),
            [pltpu.VMEM((CHUNK, EMBED_DIM), jnp.float32) for _ in range(2)],
            [pltpu.SemaphoreType.DMA for _ in range(2)],
            [pltpu.SemaphoreType.DMA for _ in range(2)],
        ],
        compiler_params=pltpu.CompilerParams(use_tc_tiling_on_sc=False),
    )
    def _sc_gather(idx_hbm, table_hbm, out_hbm, idx_v, rows, gsem, wsem):
        wid = lax.axis_index("s") * NUM_CORES + lax.axis_index("c")
        base = wid * bpw

        pltpu.sync_copy(idx_hbm.at[pl.ds(base, bpw)], idx_v)

        def out_slice(c):
            run = wid * nrw + c
            return out_hbm.at[pl.ds((run // 4) * CHUNK, CHUNK), run % 4]

        def fire_gather(c):
            p = c % 2
            pltpu.async_copy(
                table_hbm.at[idx_v.at[pl.ds(c * CHUNK, CHUNK)]], rows[p],
                gsem[p])

        def wait_gather(c):
            p = c % 2
            pltpu.make_async_copy(
                table_hbm.at[idx_v.at[pl.ds(c * CHUNK, CHUNK)]], rows[p],
                gsem[p]).wait()

        def fire_write(c):
            p = c % 2
            pltpu.async_copy(rows[p], out_slice(c), wsem[p])

        def wait_write(c):
            p = c % 2
            pltpu.make_async_copy(rows[p], out_slice(c), wsem[p]).wait()

        fire_gather(0)
        fire_gather(1)
        for c in range(nrw):
            wait_gather(c)
            fire_write(c)
            if c + 2 < nrw:
                wait_write(c)
                fire_gather(c + 2)
        wait_write(nrw - 2)
        wait_write(nrw - 1)

    return _sc_gather


NH_A = 26
NH_B = HIST - NH_A
_sc_gather_a = _make_sc_gather(NH_A * BATCH // (NW * CHUNK))
_sc_gather_b = _make_sc_gather(NH_B * BATCH // (NW * CHUNK))


GB = 4096
GR = GB // 4


def _tc_relayout_kernel(x_ref, out_ref):
    x = x_ref[...].reshape(GR, 128)
    y = x.T
    for q in range(4):
        z = y[32 * q:32 * (q + 1), :]
        for t in range(GR // 128):
            out_ref[0, :, (GR // 128) * q + t, :, :] = (
                z[:, 128 * t:128 * (t + 1)].reshape(4, 8, 128))


def _tc_relayout_alias_kernel(x_ref, prev_ref, out_ref):
    del prev_ref
    _tc_relayout_kernel(x_ref, out_ref)


_out5_type = jax.ShapeDtypeStruct((HIST, 4, BATCH // 128, 8, 128),
                                  jnp.float32)


def _out5_spec(h0):
    return pl.BlockSpec((1, 4, GB // 128, 8, 128),
                        lambda h, g: (h + h0, 0, g, 0, 0))


_tc_relayout_a = pl.pallas_call(
    _tc_relayout_kernel,
    grid=(NH_A, BATCH // GB),
    in_specs=[pl.BlockSpec((GB * EMBED_DIM,),
                           lambda h, g: ((BATCH // GB) * h + g,))],
    out_specs=_out5_spec(0),
    out_shape=_out5_type,
)

_tc_relayout_b = pl.pallas_call(
    _tc_relayout_alias_kernel,
    grid=(NH_B, BATCH // GB),
    in_specs=[pl.BlockSpec((GB * EMBED_DIM,),
                           lambda h, g: ((BATCH // GB) * h + g,)),
              _out5_spec(NH_A)],
    out_specs=_out5_spec(NH_A),
    out_shape=_out5_type,
    input_output_aliases={1: 0},
)


TVB = 8192
TGRID = pl.cdiv(VOCAB, TVB)
VOCAB_P = TGRID * TVB


def _tc_table_kernel(x_ref, o_ref):
    y = x_ref[...].T
    o_ref[...] = jnp.concatenate(
        [y[2048 * j:2048 * (j + 1), :] for j in range(4)], axis=1)


_tc_table = pl.pallas_call(
    _tc_table_kernel,
    grid=(TGRID,),
    in_specs=[pl.BlockSpec((EMBED_DIM, TVB), lambda g: (0, g))],
    out_specs=pl.BlockSpec((TVB // 4, 128), lambda g: (g, 0)),
    out_shape=jax.ShapeDtypeStruct((VOCAB_P // 4, 128), jnp.float32),
)


def kernel(input, table):
    v = input
    vr = (v & -8192) + ((v & 2047) * 4) + ((v & 8191) // 2048)
    idx = vr.T.reshape(-1)
    tbl = _tc_table(table.T).reshape(VOCAB_P, EMBED_DIM)
    split = NH_A * BATCH
    inter_a = _sc_gather_a(idx[:split], tbl)
    inter_b = _sc_gather_b(idx[split:], tbl)
    out5a = _tc_relayout_a(inter_a.reshape(-1))
    out5 = _tc_relayout_b(inter_b.reshape(-1), out5a)
    return out5.transpose(2, 4, 0, 1, 3).reshape(BATCH, HIST, EMBED_DIM)


# speedup vs baseline: 1.0002x; 1.0002x over previous
import functools

import jax
import jax.numpy as jnp
from jax import lax
from jax.experimental import pallas as pl
from jax.experimental.pallas import tpu as pltpu
from jax.experimental.pallas import tpu_sc as plsc

VOCAB = 1000000
EMBED_DIM = 32
BATCH = 16384
HIST = 50
B = BATCH * HIST

NUM_CORES = 2
NUM_SUBCORES = 16
NW = NUM_CORES * NUM_SUBCORES
BPW = B // NW

CHUNK = 1024
NCHUNK = BPW // CHUNK

_mesh = plsc.VectorSubcoreMesh(core_axis_name="c", subcore_axis_name="s")


@functools.partial(
    pl.kernel,
    out_type=jax.ShapeDtypeStruct((B // 4, 4, EMBED_DIM), jnp.float32),
    mesh=_mesh,
    scratch_types=[
        pltpu.VMEM((BPW,), jnp.int32),
        [pltpu.VMEM((CHUNK, EMBED_DIM), jnp.float32) for _ in range(2)],
        [pltpu.SemaphoreType.DMA for _ in range(2)],
        [pltpu.SemaphoreType.DMA for _ in range(2)],
    ],
    compiler_params=pltpu.CompilerParams(use_tc_tiling_on_sc=False),
)
def _sc_gather(idx_hbm, table_hbm, out_hbm, idx_v, rows, gsem, wsem):
    wid = lax.axis_index("s") * NUM_CORES + lax.axis_index("c")
    base = wid * BPW

    pltpu.sync_copy(idx_hbm.at[pl.ds(base, BPW)], idx_v)

    def out_slice(c):
        run = wid * NCHUNK + c
        return out_hbm.at[pl.ds((run // 4) * CHUNK, CHUNK), run % 4]

    def fire_gather(c):
        p = c % 2
        pltpu.async_copy(
            table_hbm.at[idx_v.at[pl.ds(c * CHUNK, CHUNK)]], rows[p], gsem[p])

    def wait_gather(c):
        p = c % 2
        pltpu.make_async_copy(
            table_hbm.at[idx_v.at[pl.ds(c * CHUNK, CHUNK)]], rows[p],
            gsem[p]).wait()

    def fire_write(c):
        p = c % 2
        pltpu.async_copy(rows[p], out_slice(c), wsem[p])

    def wait_write(c):
        p = c % 2
        pltpu.make_async_copy(rows[p], out_slice(c), wsem[p]).wait()

    fire_gather(0)
    fire_gather(1)
    for c in range(NCHUNK):
        wait_gather(c)
        fire_write(c)
        if c + 2 < NCHUNK:
            wait_write(c)
            fire_gather(c + 2)
    wait_write(NCHUNK - 2)
    wait_write(NCHUNK - 1)


GB = 4096
GR = GB // 4


def _tc_relayout_kernel(x_ref, out_ref):
    x = x_ref[...].reshape(GR, 128)
    y = x.T
    for q in range(4):
        z = y[32 * q:32 * (q + 1), :]
        for t in range(GR // 128):
            out_ref[0, :, (GR // 128) * q + t, :, :] = (
                z[:, 128 * t:128 * (t + 1)].reshape(4, 8, 128))


_tc_relayout = pl.pallas_call(
    _tc_relayout_kernel,
    grid=(HIST, BATCH // GB),
    in_specs=[pl.BlockSpec((GB * EMBED_DIM,),
                           lambda h, g: ((BATCH // GB) * h + g,))],
    out_specs=pl.BlockSpec((1, 4, GB // 128, 8, 128),
                           lambda h, g: (h, 0, g, 0, 0)),
    out_shape=jax.ShapeDtypeStruct((HIST, 4, BATCH // 128, 8, 128),
                                   jnp.float32),
)


TVB = 8192
TGRID = pl.cdiv(VOCAB, TVB)
VOCAB_P = TGRID * TVB


def _tc_table_kernel(x_ref, o_ref):
    y = x_ref[...].T
    o_ref[...] = jnp.concatenate(
        [y[2048 * j:2048 * (j + 1), :] for j in range(4)], axis=1)


_tc_table = pl.pallas_call(
    _tc_table_kernel,
    grid=(TGRID,),
    in_specs=[pl.BlockSpec((EMBED_DIM, TVB), lambda g: (0, g))],
    out_specs=pl.BlockSpec((TVB // 4, 128), lambda g: (g, 0)),
    out_shape=jax.ShapeDtypeStruct((VOCAB_P // 4, 128), jnp.float32),
)


def kernel(input, table):
    v = input
    vr = (v & -8192) + ((v & 2047) * 4) + ((v & 8191) // 2048)
    idx = vr.T.reshape(-1)
    tbl = _tc_table(table.T).reshape(VOCAB_P, EMBED_DIM)
    inter = _sc_gather(idx, tbl)
    out5 = _tc_relayout(inter.reshape(-1))
    return out5.transpose(2, 4, 0, 1, 3).reshape(BATCH, HIST, EMBED_DIM)
